# baseline (speedup 1.0000x reference)
import functools

import jax
import jax.numpy as jnp
from jax import lax
from jax.experimental import pallas as pl
from jax.experimental.pallas import tpu as pltpu
from jax.experimental.pallas import tpu_sc as plsc

NW = 32
TPW = 64
RPW = 32
NTAB = 9
SL_OFF = (0, 2048, 4096, 6144, 8192)
SL_W = (2048, 2048, 2048, 2048, 1792)
NSLICE = len(SL_OFF)
CSL = 2048
DMAIN = 9984


def _level_idx(x, low, high, n):
    xc = jnp.clip(x, low, high)
    idx = jnp.round((xc - low) / (high - low) * (n - 1)).astype(jnp.int32)
    return jnp.clip(idx, 0, n - 1)


def _all_indices(input, n_lvl, n_time):
    SIGNAL_MIN, SIGNAL_MAX = -5.0, 5.0
    MAG_MIN, MAG_MAX = -10.0, 10.0
    ENERGY_MIN, ENERGY_MAX = -10.0, 10.0

    t = input[:, 0] - input[0, 0]
    xyz = input[:, 1:]

    idx_x = _level_idx(jnp.clip(xyz[:, 0], SIGNAL_MIN, SIGNAL_MAX), SIGNAL_MIN, SIGNAL_MAX, n_lvl)
    idx_y = _level_idx(jnp.clip(xyz[:, 1], SIGNAL_MIN, SIGNAL_MAX), SIGNAL_MIN, SIGNAL_MAX, n_lvl)
    idx_z = _level_idx(jnp.clip(xyz[:, 2], SIGNAL_MIN, SIGNAL_MAX), SIGNAL_MIN, SIGNAL_MAX, n_lvl)

    mags = jnp.sqrt(jnp.sum(jnp.square(xyz), axis=1))
    idx_mag = _level_idx(mags, MAG_MIN, MAG_MAX, n_lvl)

    dt = t[1:] - t[:-1]
    jerk_body = (xyz[1:] - xyz[:-1]) / dt[:, None]
    jerk = jnp.concatenate([jnp.zeros((1, 3), dtype=input.dtype), jerk_body], axis=0)

    idx_xj = _level_idx(jnp.clip(jerk[:, 0], SIGNAL_MIN, SIGNAL_MAX), SIGNAL_MIN, SIGNAL_MAX, n_lvl)
    idx_yj = _level_idx(jnp.clip(jerk[:, 1], SIGNAL_MIN, SIGNAL_MAX), SIGNAL_MIN, SIGNAL_MAX, n_lvl)
    idx_zj = _level_idx(jnp.clip(jerk[:, 2], SIGNAL_MIN, SIGNAL_MAX), SIGNAL_MIN, SIGNAL_MAX, n_lvl)

    jerk_mags = jnp.sqrt(jnp.sum(jnp.square(jerk), axis=1))
    idx_magj = _level_idx(jerk_mags, MAG_MIN, MAG_MAX, n_lvl)

    idx_time = _level_idx(t, 0.0, float(n_time), n_time)

    energy = jnp.sum(jnp.square(xyz), axis=0) / xyz.shape[0]
    T = input.shape[0]
    idx_ex = jnp.full((T,), _level_idx(energy[0], ENERGY_MIN, ENERGY_MAX, n_lvl), jnp.int32)
    idx_ey = jnp.full((T,), _level_idx(energy[1], ENERGY_MIN, ENERGY_MAX, n_lvl), jnp.int32)
    idx_ez = jnp.full((T,), _level_idx(energy[2], ENERGY_MIN, ENERGY_MAX, n_lvl), jnp.int32)

    return jnp.stack([idx_x, idx_y, idx_z, idx_mag, idx_xj, idx_yj, idx_zj,
                      idx_magj, idx_time, idx_ex, idx_ey, idx_ez], axis=0)


def _sc_gather_product(idx5, tabs_v):
    mesh = plsc.VectorSubcoreMesh(core_axis_name="c", subcore_axis_name="s")

    @functools.partial(
        pl.kernel,
        mesh=mesh,
        out_type=jax.ShapeDtypeStruct((NW, NSLICE, CSL), jnp.float32),
        scratch_types=[
            pltpu.VMEM((NTAB, RPW, 2), jnp.int32),
            pltpu.VMEM((2, NTAB, 2, CSL), jnp.float32),
            pltpu.VMEM((NSLICE, CSL), jnp.float32),
            pltpu.SemaphoreType.DMA,
            pltpu.SemaphoreType.DMA,
        ],
    )
    def sc_k(idx_hbm, t0, t1, t2, t3, t4, t5, t6, t7, t8, out_hbm,
             idx_v, buf, acc, sem0, sem1):
        tabs = (t0, t1, t2, t3, t4, t5, t6, t7, t8)
        sems = (sem0, sem1)
        wid = lax.axis_index("s") * 2 + lax.axis_index("c")

        pltpu.sync_copy(idx_hbm.at[wid], idx_v)

        for s in range(NSLICE):
            off, w = SL_OFF[s], SL_W[s]

            def zero_g(g, carry):
                acc[s, pl.ds(g * 16, 16)] = jnp.zeros((16,), jnp.float32)
                return carry

            lax.fori_loop(0, w // 16, zero_g, 0)

            def src(k, r):
                return tabs[k].at[idx_v.at[k, r], pl.ds(off, w)]

            def dst(p, k):
                return buf.at[p, k, :, pl.ds(0, w)]

            for p in (0, 1):
                for k in range(NTAB):
                    pltpu.async_copy(src(k, p), dst(p, k), sems[p])

            def round_pair(i2, carry):
                for p in (0, 1):
                    r = i2 * 2 + p
                    for k in range(NTAB):
                        pltpu.make_async_copy(src(k, r), dst(p, k),
                                              sems[p]).wait()

                    def grp(g, c2):
                        col = pl.ds(g * 16, 16)
                        v0 = buf[p, 0, 0, col]
                        v1 = buf[p, 0, 1, col]
                        for k in range(1, NTAB):
                            v0 = v0 * buf[p, k, 0, col]
                            v1 = v1 * buf[p, k, 1, col]
                        acc[s, col] = acc[s, col] + (v0 + v1)
                        return c2

                    lax.fori_loop(0, w // 16, grp, 0)

                    rn = r + 2

                    @pl.when(rn < RPW)
                    def _():
                        for k in range(NTAB):
                            pltpu.async_copy(src(k, rn), dst(p, k), sems[p])
                return carry

            lax.fori_loop(0, RPW // 2, round_pair, 0)

        pltpu.sync_copy(acc, out_hbm.at[wid])

    return sc_k(idx5, *tabs_v)


def _finale_body(eidx_ref, parts, tail, ex, ey, ez, out_ref):
    s = jnp.sum(parts[...], axis=0, keepdims=True)
    full = jnp.concatenate([s[:, :DMAIN], tail[:, :16]], axis=1)
    out_ref[...] = jnp.tanh(full * (ex[0] * ey[0] * ez[0]))


def _tc_finale(partials, tail, T_ex, T_ey, T_ez, e_idx, D):
    n = T_ex.shape[0]
    tabs = tuple(t.reshape(n, 1, D) for t in (T_ex, T_ey, T_ez))
    PW = NSLICE * CSL
    grid_spec = pltpu.PrefetchScalarGridSpec(
        num_scalar_prefetch=1,
        grid=(1,),
        in_specs=[pl.BlockSpec((NW, PW), lambda i, e: (0, 0)),
                  pl.BlockSpec((1, 128), lambda i, e: (0, 0))]
        + [pl.BlockSpec((1, 1, D), lambda i, e, k=k: (e[k], 0, 0)) for k in range(3)],
        out_specs=pl.BlockSpec((1, D), lambda i, e: (0, 0)),
    )
    out = pl.pallas_call(
        _finale_body,
        grid_spec=grid_spec,
        out_shape=jax.ShapeDtypeStruct((1, D), jnp.float32),
    )(e_idx, partials, tail, *tabs)
    return out[0]


def kernel(input, T_x, T_y, T_z, T_mag, T_xj, T_yj, T_zj, T_magj, T_ex, T_ey, T_ez, T_time):
    n_lvl = T_x.shape[0]
    n_time = T_time.shape[0]
    D = T_x.shape[1]
    T = input.shape[0]

    idx_all = _all_indices(input, n_lvl, n_time)

    idx9 = idx_all[:NTAB].reshape(NTAB, NW, RPW, 2)
    idx9 = jnp.transpose(idx9, (1, 0, 2, 3)).astype(jnp.int32)

    tabs = (T_x, T_y, T_z, T_mag, T_xj, T_yj, T_zj, T_magj, T_time)
    partials = _sc_gather_product(idx9, tabs).reshape(NW, NSLICE * CSL)

    tail16 = tabs[0][idx_all[0], DMAIN:]
    for k in range(1, NTAB):
        tail16 = tail16 * tabs[k][idx_all[k], DMAIN:]
    tail = jnp.zeros((1, 128), jnp.float32).at[0, :16].set(jnp.sum(tail16, axis=0))

    out = _tc_finale(partials, tail, T_ex, T_ey, T_ez, idx_all[NTAB:, 0], D)
    return out


# speedup vs baseline: 56.2128x; 56.2128x over previous
import functools

import jax
import jax.numpy as jnp
from jax import lax
from jax.experimental import pallas as pl
from jax.experimental.pallas import tpu as pltpu
from jax.experimental.pallas import tpu_sc as plsc

NW = 32
TPW = 64
RPW = 32
NTAB = 9
SL_OFF = (0, 2048, 4096, 6144, 8192)
SL_W = (2048, 2048, 2048, 2048, 1792)
NSLICE = len(SL_OFF)
CSL = 2048
DMAIN = 9984


def _level_idx(x, low, high, n):
    xc = jnp.clip(x, low, high)
    idx = jnp.round((xc - low) / (high - low) * (n - 1)).astype(jnp.int32)
    return jnp.clip(idx, 0, n - 1)


def _all_indices(input, n_lvl, n_time):
    SIGNAL_MIN, SIGNAL_MAX = -5.0, 5.0
    MAG_MIN, MAG_MAX = -10.0, 10.0
    ENERGY_MIN, ENERGY_MAX = -10.0, 10.0

    t = input[:, 0] - input[0, 0]
    xyz = input[:, 1:]

    idx_x = _level_idx(jnp.clip(xyz[:, 0], SIGNAL_MIN, SIGNAL_MAX), SIGNAL_MIN, SIGNAL_MAX, n_lvl)
    idx_y = _level_idx(jnp.clip(xyz[:, 1], SIGNAL_MIN, SIGNAL_MAX), SIGNAL_MIN, SIGNAL_MAX, n_lvl)
    idx_z = _level_idx(jnp.clip(xyz[:, 2], SIGNAL_MIN, SIGNAL_MAX), SIGNAL_MIN, SIGNAL_MAX, n_lvl)

    mags = jnp.sqrt(jnp.sum(jnp.square(xyz), axis=1))
    idx_mag = _level_idx(mags, MAG_MIN, MAG_MAX, n_lvl)

    dt = t[1:] - t[:-1]
    jerk_body = (xyz[1:] - xyz[:-1]) / dt[:, None]
    jerk = jnp.concatenate([jnp.zeros((1, 3), dtype=input.dtype), jerk_body], axis=0)

    idx_xj = _level_idx(jnp.clip(jerk[:, 0], SIGNAL_MIN, SIGNAL_MAX), SIGNAL_MIN, SIGNAL_MAX, n_lvl)
    idx_yj = _level_idx(jnp.clip(jerk[:, 1], SIGNAL_MIN, SIGNAL_MAX), SIGNAL_MIN, SIGNAL_MAX, n_lvl)
    idx_zj = _level_idx(jnp.clip(jerk[:, 2], SIGNAL_MIN, SIGNAL_MAX), SIGNAL_MIN, SIGNAL_MAX, n_lvl)

    jerk_mags = jnp.sqrt(jnp.sum(jnp.square(jerk), axis=1))
    idx_magj = _level_idx(jerk_mags, MAG_MIN, MAG_MAX, n_lvl)

    idx_time = _level_idx(t, 0.0, float(n_time), n_time)

    energy = jnp.sum(jnp.square(xyz), axis=0) / xyz.shape[0]
    T = input.shape[0]
    idx_ex = jnp.full((T,), _level_idx(energy[0], ENERGY_MIN, ENERGY_MAX, n_lvl), jnp.int32)
    idx_ey = jnp.full((T,), _level_idx(energy[1], ENERGY_MIN, ENERGY_MAX, n_lvl), jnp.int32)
    idx_ez = jnp.full((T,), _level_idx(energy[2], ENERGY_MIN, ENERGY_MAX, n_lvl), jnp.int32)

    return jnp.stack([idx_x, idx_y, idx_z, idx_mag, idx_xj, idx_yj, idx_zj,
                      idx_magj, idx_time, idx_ex, idx_ey, idx_ez], axis=0)


def _sc_gather_product(idx5, tabs_v):
    mesh = plsc.VectorSubcoreMesh(core_axis_name="c", subcore_axis_name="s")

    @functools.partial(
        pl.kernel,
        mesh=mesh,
        out_type=jax.ShapeDtypeStruct((NW, NSLICE, CSL), jnp.float32),
        scratch_types=[
            pltpu.VMEM((NTAB, RPW, 2), jnp.int32),
            pltpu.VMEM((2, NTAB, 2, CSL), jnp.float32),
            pltpu.VMEM((NSLICE, CSL), jnp.float32),
            pltpu.SemaphoreType.DMA,
            pltpu.SemaphoreType.DMA,
        ],
    )
    def sc_k(idx_hbm, t0, t1, t2, t3, t4, t5, t6, t7, t8, out_hbm,
             idx_v, buf, acc, sem0, sem1):
        tabs = (t0, t1, t2, t3, t4, t5, t6, t7, t8)
        sems = (sem0, sem1)
        wid = lax.axis_index("s") * 2 + lax.axis_index("c")

        pltpu.sync_copy(idx_hbm.at[wid], idx_v)

        for s in range(NSLICE):
            off, w = SL_OFF[s], SL_W[s]

            def zero_g(g, carry):
                acc[s, pl.ds(g * 16, 16)] = jnp.zeros((16,), jnp.float32)
                return carry

            lax.fori_loop(0, w // 16, zero_g, 0)

            def src(k, r):
                return tabs[k].at[idx_v.at[k, r], pl.ds(off, w)]

            def dst(p, k):
                return buf.at[p, k, :, pl.ds(0, w)]

            for p in (0, 1):
                for k in range(NTAB):
                    pltpu.async_copy(src(k, p), dst(p, k), sems[p])

            def round_pair(i2, carry):
                for p in (0, 1):
                    r = i2 * 2 + p
                    for k in range(NTAB):
                        pltpu.make_async_copy(src(k, r), dst(p, k),
                                              sems[p]).wait()

                    def grp(g, c2):
                        col = pl.ds(g * 16, 16)
                        v0 = buf[p, 0, 0, col]
                        v1 = buf[p, 0, 1, col]
                        for k in range(1, NTAB):
                            v0 = v0 * buf[p, k, 0, col]
                            v1 = v1 * buf[p, k, 1, col]
                        acc[s, col] = acc[s, col] + (v0 + v1)
                        return c2

                    lax.fori_loop(0, w // 16, grp, 0)

                    rn = r + 2

                    @pl.when(rn < RPW)
                    def _():
                        for k in range(NTAB):
                            pltpu.async_copy(src(k, rn), dst(p, k), sems[p])
                return carry

            lax.fori_loop(0, RPW // 2, round_pair, 0)

        pltpu.sync_copy(acc, out_hbm.at[wid])

    return sc_k(idx5, *tabs_v)


def _finale_body(eidx_ref, parts, tail, ex, ey, ez, out_ref):
    s = jnp.sum(parts[...], axis=0, keepdims=True)
    full = jnp.concatenate([s[:, :DMAIN], tail[:, :16]], axis=1)
    out_ref[...] = jnp.tanh(full * (ex[0] * ey[0] * ez[0]))


def _tc_finale(partials, tail, T_ex, T_ey, T_ez, e_idx, D):
    n = T_ex.shape[0]
    tabs = tuple(t.reshape(n, 1, D) for t in (T_ex, T_ey, T_ez))
    PW = NSLICE * CSL
    grid_spec = pltpu.PrefetchScalarGridSpec(
        num_scalar_prefetch=1,
        grid=(1,),
        in_specs=[pl.BlockSpec((NW, PW), lambda i, e: (0, 0)),
                  pl.BlockSpec((1, 128), lambda i, e: (0, 0))]
        + [pl.BlockSpec((1, 1, D), lambda i, e, k=k: (e[k], 0, 0)) for k in range(3)],
        out_specs=pl.BlockSpec((1, D), lambda i, e: (0, 0)),
    )
    out = pl.pallas_call(
        _finale_body,
        grid_spec=grid_spec,
        out_shape=jax.ShapeDtypeStruct((1, D), jnp.float32),
    )(e_idx, partials, tail, *tabs)
    return out[0]


def kernel(input, T_x, T_y, T_z, T_mag, T_xj, T_yj, T_zj, T_magj, T_ex, T_ey, T_ez, T_time):
    n_lvl = T_x.shape[0]
    n_time = T_time.shape[0]
    D = T_x.shape[1]
    T = input.shape[0]

    idx_all = _all_indices(input, n_lvl, n_time)

    idx9 = idx_all[:NTAB].reshape(NTAB, NW, RPW, 2)
    idx9 = jnp.transpose(idx9, (1, 0, 2, 3)).astype(jnp.int32)

    tabs = (T_x, T_y, T_z, T_mag, T_xj, T_yj, T_zj, T_magj, T_time)
    partials = _sc_gather_product(idx9, tabs).reshape(NW, NSLICE * CSL)

    tail16 = jnp.take(tabs[0][:, DMAIN:], idx_all[0], axis=0)
    for k in range(1, NTAB):
        tail16 = tail16 * jnp.take(tabs[k][:, DMAIN:], idx_all[k], axis=0)
    tail = jnp.zeros((1, 128), jnp.float32).at[0, :16].set(jnp.sum(tail16, axis=0))

    out = _tc_finale(partials, tail, T_ex, T_ey, T_ez, idx_all[NTAB:, 0], D)
    return out
